# baseline (speedup 1.0000x reference)
import functools

import jax
import jax.numpy as jnp
from jax import lax
from jax.experimental import pallas as pl
from jax.experimental.pallas import tpu as pltpu
from jax.experimental.pallas import tpu_sc as plsc

B = 8
Q = 20000
T = 100
TPAD = 112
NCORE = 2
NSUB = 16
PER_W = 1280
CHUNKS = PER_W // 16
B_PER_CORE = B // NCORE
G = 4


def _matcher_body(pred_hbm, tgt_hbm, outmax_hbm, outidx_hbm,
                  praw, x0a, y0a, x1a, y1a, aa, traw,
                  lmax, lidx, mgmax, mgidx, omax, oidx,
                  shmax, shidx):
    c = lax.axis_index("c")
    s = lax.axis_index("s")
    iota = lax.iota(jnp.int32, 16)
    lane0 = iota == 0
    zc = jnp.zeros((16,), jnp.int32)
    base = jnp.minimum(s * PER_W, Q - PER_W)

    for bl in range(B_PER_CORE):
        b = c * B_PER_CORE + bl
        pltpu.sync_copy(pred_hbm.at[pl.ds(b * (Q * 4) + base * 4, PER_W * 4)], praw)
        pltpu.sync_copy(tgt_hbm.at[pl.ds(b * (T * 4), T * 4)], traw.at[pl.ds(0, T * 4)])

        def pre(j, _):
            r = j * 64 + iota * 4
            cx = plsc.load_gather(praw, [r])
            cy = plsc.load_gather(praw, [r + 1])
            w = plsc.load_gather(praw, [r + 2])
            h = plsc.load_gather(praw, [r + 3])
            x0 = cx - 0.5 * w
            y0 = cy - 0.5 * h
            x1 = cx + 0.5 * w
            y1 = cy + 0.5 * h
            sl = pl.ds(j * 16, 16)
            x0a[sl] = x0
            y0a[sl] = y0
            x1a[sl] = x1
            y1a[sl] = y1
            aa[sl] = (x1 - x0) * (y1 - y0)
            return 0

        lax.fori_loop(0, CHUNKS, pre, 0)

        def per_g(g, _):
            t0 = g * G
            tco = []
            for i in range(G):
                t4 = zc + (t0 + i) * 4
                tcx = plsc.load_gather(traw, [t4])
                tcy = plsc.load_gather(traw, [t4 + 1])
                tw = plsc.load_gather(traw, [t4 + 2])
                th = plsc.load_gather(traw, [t4 + 3])
                tx0 = tcx - 0.5 * tw
                ty0 = tcy - 0.5 * th
                tx1 = tcx + 0.5 * tw
                ty1 = tcy + 0.5 * th
                ta = (tx1 - tx0) * (ty1 - ty0)
                tco.append((tx0, ty0, tx1, ty1, ta))

            def scan_k(k, carry):
                ms, bis, idxv = carry
                sl = pl.ds(k * 16, 16)
                x0 = x0a[sl]
                y0 = y0a[sl]
                x1 = x1a[sl]
                y1 = y1a[sl]
                av = aa[sl]
                nms, nbis = [], []
                for i in range(G):
                    tx0, ty0, tx1, ty1, ta = tco[i]
                    ltx = jnp.maximum(x0, tx0)
                    lty = jnp.maximum(y0, ty0)
                    rbx = jnp.minimum(x1, tx1)
                    rby = jnp.minimum(y1, ty1)
                    inter = jnp.maximum(rbx - ltx, 0.0) * jnp.maximum(rby - lty, 0.0)
                    union = av + ta - inter
                    iou = inter / jnp.maximum(union, 1e-9)
                    lcx = jnp.minimum(x0, tx0)
                    lcy = jnp.minimum(y0, ty0)
                    rcx = jnp.maximum(x1, tx1)
                    rcy = jnp.maximum(y1, ty1)
                    areac = jnp.maximum(rcx - lcx, 0.0) * jnp.maximum(rcy - lcy, 0.0)
                    gv = iou - (areac - union) / jnp.maximum(areac, 1e-9)
                    upd = gv > ms[i]
                    nms.append(jnp.where(upd, gv, ms[i]))
                    nbis.append(jnp.where(upd, idxv, bis[i]))
                return tuple(nms), tuple(nbis), idxv + 16

            m0 = jnp.full((16,), -3.0e38, jnp.float32)
            bi0 = jnp.zeros((16,), jnp.int32)
            ms, bis, _ = lax.fori_loop(
                0, CHUNKS, scan_k,
                ((m0,) * G, (bi0,) * G, base + iota))
            for i in range(G):
                gm = jnp.max(ms[i])
                cand = jnp.where(ms[i] == jnp.full((16,), gm),
                                 bis[i], jnp.full((16,), 1 << 30, jnp.int32))
                gi = jnp.min(cand)
                posv = zc + (bl * TPAD + t0 + i)
                plsc.store_scatter(lmax, [posv], jnp.full((16,), gm), mask=lane0)
                plsc.store_scatter(lidx, [posv], jnp.full((16,), gi, jnp.int32),
                                   mask=lane0)
            return 0

        lax.fori_loop(0, TPAD // G, per_g, 0)

    pltpu.sync_copy(lmax, shmax.at[pl.ds(s * (B_PER_CORE * TPAD), B_PER_CORE * TPAD)])
    pltpu.sync_copy(lidx, shidx.at[pl.ds(s * (B_PER_CORE * TPAD), B_PER_CORE * TPAD)])
    plsc.subcore_barrier()

    @pl.when(s < B_PER_CORE)
    def _merge():
        for w in range(NSUB):
            pltpu.sync_copy(shmax.at[pl.ds(w * (B_PER_CORE * TPAD) + s * TPAD, TPAD)],
                            mgmax.at[pl.ds(w * TPAD, TPAD)])
            pltpu.sync_copy(shidx.at[pl.ds(w * (B_PER_CORE * TPAD) + s * TPAD, TPAD)],
                            mgidx.at[pl.ds(w * TPAD, TPAD)])

        def mg(cc, _):
            sl = pl.ds(cc * 16, 16)
            del _
            acc = mgmax[sl]
            acci = mgidx[sl]
            for w in range(1, NSUB):
                wsl = pl.ds(w * TPAD + cc * 16, 16)
                v = mgmax[wsl]
                vi = mgidx[wsl]
                upd = (v > acc) | ((v == acc) & (vi < acci))
                acc = jnp.where(upd, v, acc)
                acci = jnp.where(upd, vi, acci)
            omax[sl] = acc
            oidx[sl] = acci
            return 0

        lax.fori_loop(0, TPAD // 16, mg, 0)
        gb = c * B_PER_CORE + s
        pltpu.sync_copy(omax, outmax_hbm.at[pl.ds(gb * TPAD, TPAD)])
        pltpu.sync_copy(oidx, outidx_hbm.at[pl.ds(gb * TPAD, TPAD)])


@jax.jit
def _matcher(pred_boxes, target_boxes):
    f = pl.kernel(
        _matcher_body,
        out_type=[
            jax.ShapeDtypeStruct((B * TPAD,), jnp.float32),
            jax.ShapeDtypeStruct((B * TPAD,), jnp.int32),
        ],
        mesh=plsc.VectorSubcoreMesh(core_axis_name="c", subcore_axis_name="s",
                                    num_cores=NCORE, num_subcores=NSUB),
        compiler_params=pltpu.CompilerParams(needs_layout_passes=False),
        scratch_types=[
            pltpu.VMEM((PER_W * 4,), jnp.float32),
            pltpu.VMEM((PER_W,), jnp.float32),
            pltpu.VMEM((PER_W,), jnp.float32),
            pltpu.VMEM((PER_W,), jnp.float32),
            pltpu.VMEM((PER_W,), jnp.float32),
            pltpu.VMEM((PER_W,), jnp.float32),
            pltpu.VMEM((TPAD * 4,), jnp.float32),
            pltpu.VMEM((B_PER_CORE * TPAD,), jnp.float32),
            pltpu.VMEM((B_PER_CORE * TPAD,), jnp.int32),
            pltpu.VMEM((NSUB * TPAD,), jnp.float32),
            pltpu.VMEM((NSUB * TPAD,), jnp.int32),
            pltpu.VMEM((TPAD,), jnp.float32),
            pltpu.VMEM((TPAD,), jnp.int32),
            pltpu.VMEM_SHARED((NSUB * B_PER_CORE * TPAD,), jnp.float32),
            pltpu.VMEM_SHARED((NSUB * B_PER_CORE * TPAD,), jnp.int32),
        ],
    )
    om, oi = f(pred_boxes.reshape(B * Q * 4), target_boxes.reshape(B * T * 4))
    return om.reshape(B, TPAD), oi.reshape(B, TPAD)


def kernel(pred_boxes, target_boxes):
    outmax, outidx = _matcher(pred_boxes, target_boxes)
    max_iou = outmax[:, :T]
    pred_idx = outidx[:, :T]
    valid = max_iou >= 0.5
    return pred_idx, valid, max_iou


# speedup vs baseline: 1.0735x; 1.0735x over previous
import functools

import jax
import jax.numpy as jnp
from jax import lax
from jax.experimental import pallas as pl
from jax.experimental.pallas import tpu as pltpu
from jax.experimental.pallas import tpu_sc as plsc

B = 8
Q = 20000
T = 100
TPAD = 112
NCORE = 2
NSUB = 16
PER_W = 1280
CHUNKS = PER_W // 16
B_PER_CORE = B // NCORE
G = 4


def _matcher_body(pred_hbm, tgt_hbm, outmax_hbm, outidx_hbm,
                  praw, x0a, y0a, x1a, y1a, aa, traw,
                  lmax, lidx, mgmax, mgidx, omax, oidx,
                  shmax, shidx):
    c = lax.axis_index("c")
    s = lax.axis_index("s")
    iota = lax.iota(jnp.int32, 16)
    lane0 = iota == 0
    zc = jnp.zeros((16,), jnp.int32)
    base = jnp.minimum(s * PER_W, Q - PER_W)

    for bl in range(B_PER_CORE):
        b = c * B_PER_CORE + bl
        pltpu.sync_copy(pred_hbm.at[pl.ds(b * (Q * 4) + base * 4, PER_W * 4)], praw)
        pltpu.sync_copy(tgt_hbm.at[pl.ds(b * (T * 4), T * 4)], traw.at[pl.ds(0, T * 4)])

        def pre(j, _):
            r = j * 64 + iota * 4
            cx = plsc.load_gather(praw, [r])
            cy = plsc.load_gather(praw, [r + 1])
            w = plsc.load_gather(praw, [r + 2])
            h = plsc.load_gather(praw, [r + 3])
            x0 = cx - 0.5 * w
            y0 = cy - 0.5 * h
            x1 = cx + 0.5 * w
            y1 = cy + 0.5 * h
            sl = pl.ds(j * 16, 16)
            x0a[sl] = x0
            y0a[sl] = y0
            x1a[sl] = x1
            y1a[sl] = y1
            aa[sl] = (x1 - x0) * (y1 - y0)
            return 0

        lax.fori_loop(0, CHUNKS, pre, 0)

        def per_g(g, _):
            t0 = g * G
            tco = []
            for i in range(G):
                t4 = zc + (t0 + i) * 4
                tcx = plsc.load_gather(traw, [t4])
                tcy = plsc.load_gather(traw, [t4 + 1])
                tw = plsc.load_gather(traw, [t4 + 2])
                th = plsc.load_gather(traw, [t4 + 3])
                tx0 = tcx - 0.5 * tw
                ty0 = tcy - 0.5 * th
                tx1 = tcx + 0.5 * tw
                ty1 = tcy + 0.5 * th
                ta = (tx1 - tx0) * (ty1 - ty0)
                tco.append((tx0, ty0, tx1, ty1, ta))

            def scan_k(k, carry):
                ms, bis, idxv = carry
                sl = pl.ds(k * 16, 16)
                x0 = x0a[sl]
                y0 = y0a[sl]
                x1 = x1a[sl]
                y1 = y1a[sl]
                av = aa[sl]
                nms, nbis = [], []
                for i in range(G):
                    tx0, ty0, tx1, ty1, ta = tco[i]
                    ltx = jnp.maximum(x0, tx0)
                    lty = jnp.maximum(y0, ty0)
                    rbx = jnp.minimum(x1, tx1)
                    rby = jnp.minimum(y1, ty1)
                    inter = jnp.maximum(rbx - ltx, 0.0) * jnp.maximum(rby - lty, 0.0)
                    union = av + ta - inter
                    iou = inter / jnp.maximum(union, 1e-9)
                    lcx = jnp.minimum(x0, tx0)
                    lcy = jnp.minimum(y0, ty0)
                    rcx = jnp.maximum(x1, tx1)
                    rcy = jnp.maximum(y1, ty1)
                    areac = jnp.maximum(rcx - lcx, 0.0) * jnp.maximum(rcy - lcy, 0.0)
                    gv = iou - (areac - union) / jnp.maximum(areac, 1e-9)
                    upd = gv > ms[i]
                    nms.append(jnp.where(upd, gv, ms[i]))
                    nbis.append(jnp.where(upd, idxv, bis[i]))
                return tuple(nms), tuple(nbis), idxv + 16

            m0 = jnp.full((16,), -3.0e38, jnp.float32)
            bi0 = jnp.zeros((16,), jnp.int32)
            ms, bis, _ = lax.fori_loop(
                0, CHUNKS, scan_k,
                ((m0,) * G, (bi0,) * G, base + iota))
            for i in range(G):
                gm = jnp.max(ms[i])
                cand = jnp.where(ms[i] == jnp.full((16,), gm),
                                 bis[i], jnp.full((16,), 1 << 30, jnp.int32))
                gi = jnp.min(cand)
                posv = zc + (bl * TPAD + t0 + i)
                plsc.store_scatter(lmax, [posv], jnp.full((16,), gm), mask=lane0)
                plsc.store_scatter(lidx, [posv], jnp.full((16,), gi, jnp.int32),
                                   mask=lane0)
            return 0

        lax.fori_loop(0, T // G, per_g, 0)

    pltpu.sync_copy(lmax, shmax.at[pl.ds(s * (B_PER_CORE * TPAD), B_PER_CORE * TPAD)])
    pltpu.sync_copy(lidx, shidx.at[pl.ds(s * (B_PER_CORE * TPAD), B_PER_CORE * TPAD)])
    plsc.subcore_barrier()

    @pl.when(s < B_PER_CORE)
    def _merge():
        for w in range(NSUB):
            pltpu.sync_copy(shmax.at[pl.ds(w * (B_PER_CORE * TPAD) + s * TPAD, TPAD)],
                            mgmax.at[pl.ds(w * TPAD, TPAD)])
            pltpu.sync_copy(shidx.at[pl.ds(w * (B_PER_CORE * TPAD) + s * TPAD, TPAD)],
                            mgidx.at[pl.ds(w * TPAD, TPAD)])

        def mg(cc, _):
            sl = pl.ds(cc * 16, 16)
            del _
            acc = mgmax[sl]
            acci = mgidx[sl]
            for w in range(1, NSUB):
                wsl = pl.ds(w * TPAD + cc * 16, 16)
                v = mgmax[wsl]
                vi = mgidx[wsl]
                upd = (v > acc) | ((v == acc) & (vi < acci))
                acc = jnp.where(upd, v, acc)
                acci = jnp.where(upd, vi, acci)
            omax[sl] = acc
            oidx[sl] = acci
            return 0

        lax.fori_loop(0, TPAD // 16, mg, 0)
        gb = c * B_PER_CORE + s
        pltpu.sync_copy(omax, outmax_hbm.at[pl.ds(gb * TPAD, TPAD)])
        pltpu.sync_copy(oidx, outidx_hbm.at[pl.ds(gb * TPAD, TPAD)])


@jax.jit
def _matcher(pred_boxes, target_boxes):
    f = pl.kernel(
        _matcher_body,
        out_type=[
            jax.ShapeDtypeStruct((B * TPAD,), jnp.float32),
            jax.ShapeDtypeStruct((B * TPAD,), jnp.int32),
        ],
        mesh=plsc.VectorSubcoreMesh(core_axis_name="c", subcore_axis_name="s",
                                    num_cores=NCORE, num_subcores=NSUB),
        compiler_params=pltpu.CompilerParams(needs_layout_passes=False),
        scratch_types=[
            pltpu.VMEM((PER_W * 4,), jnp.float32),
            pltpu.VMEM((PER_W,), jnp.float32),
            pltpu.VMEM((PER_W,), jnp.float32),
            pltpu.VMEM((PER_W,), jnp.float32),
            pltpu.VMEM((PER_W,), jnp.float32),
            pltpu.VMEM((PER_W,), jnp.float32),
            pltpu.VMEM((TPAD * 4,), jnp.float32),
            pltpu.VMEM((B_PER_CORE * TPAD,), jnp.float32),
            pltpu.VMEM((B_PER_CORE * TPAD,), jnp.int32),
            pltpu.VMEM((NSUB * TPAD,), jnp.float32),
            pltpu.VMEM((NSUB * TPAD,), jnp.int32),
            pltpu.VMEM((TPAD,), jnp.float32),
            pltpu.VMEM((TPAD,), jnp.int32),
            pltpu.VMEM_SHARED((NSUB * B_PER_CORE * TPAD,), jnp.float32),
            pltpu.VMEM_SHARED((NSUB * B_PER_CORE * TPAD,), jnp.int32),
        ],
    )
    om, oi = f(pred_boxes.reshape(B * Q * 4), target_boxes.reshape(B * T * 4))
    return om.reshape(B, TPAD), oi.reshape(B, TPAD)


def kernel(pred_boxes, target_boxes):
    outmax, outidx = _matcher(pred_boxes, target_boxes)
    max_iou = outmax[:, :T]
    pred_idx = outidx[:, :T]
    valid = max_iou >= 0.5
    return pred_idx, valid, max_iou
